# baseline (speedup 1.0000x reference)
import functools

import jax
import jax.numpy as jnp
from jax import lax
from jax.experimental import pallas as pl
from jax.experimental.pallas import tpu as pltpu
from jax.experimental.pallas import tpu_sc as plsc

N = 10000
E = 320000
D = 128
NP = 10240

NC = 2
NS = 16
NW = NC * NS
EPW = E // NW
B = 128
CHT = 80
PADR = CHT
RPT = NP // NS

def _sc_body(with_deg, x_hbm, src_hbm, dst_hbm, z2_hbm, z1_hbm,
             out_hbm, deg_hbm, src_v, dst_v, rows_v, ones_v,
             acc_s, deg_s, sem):
    c = lax.axis_index("c")
    s = lax.axis_index("s")
    wid = s * NC + c
    row0 = s * RPT

    pltpu.sync_copy(z2_hbm, acc_s.at[pl.ds(row0, RPT)])
    if with_deg:
        pltpu.sync_copy(z1_hbm, deg_s.at[pl.ds(row0, RPT)])
        for i in range(B // 16):
            ones_v[pl.ds(i * 16, 16)] = jnp.ones((16,), jnp.float32)

    pltpu.sync_copy(src_hbm.at[wid], src_v)
    pltpu.sync_copy(dst_hbm.at[wid], dst_v)
    plsc.subcore_barrier()

    def chunk(i, carry):
        pltpu.async_copy(x_hbm.at[src_v.at[i]], rows_v, sem).wait()
        pltpu.sync_copy(rows_v, acc_s.at[dst_v.at[i]], add=True)
        if with_deg:
            pltpu.sync_copy(ones_v, deg_s.at[dst_v.at[i]], add=True)
        return carry

    lax.fori_loop(0, CHT, chunk, 0)
    plsc.subcore_barrier()

    out0 = c * NP + row0
    pltpu.sync_copy(acc_s.at[pl.ds(row0, RPT)], out_hbm.at[pl.ds(out0, RPT)])
    if with_deg:
        pltpu.sync_copy(deg_s.at[pl.ds(row0, RPT)], deg_hbm.at[pl.ds(out0, RPT)])


@functools.lru_cache(maxsize=None)
def _make_sc_pass(with_deg):
    mesh = plsc.VectorSubcoreMesh(core_axis_name="c", subcore_axis_name="s")
    out_type = [jax.ShapeDtypeStruct((NC * NP, D), jnp.float32)]
    if with_deg:
        out_type.append(jax.ShapeDtypeStruct((NC * NP,), jnp.float32))
    kern = functools.partial(
        pl.kernel,
        mesh=mesh,
        out_type=out_type,
        scratch_types=[
            pltpu.VMEM((CHT, B), jnp.int32),
            pltpu.VMEM((CHT, B), jnp.int32),
            pltpu.VMEM((B, D), jnp.float32),
            pltpu.VMEM((B,), jnp.float32),
            pltpu.VMEM_SHARED((NP, D), jnp.float32),
            pltpu.VMEM_SHARED((NP,), jnp.float32),
            pltpu.SemaphoreType.DMA,
        ],
    )

    if with_deg:
        @kern
        def sc_pass(x_hbm, src_hbm, dst_hbm, z2_hbm, z1_hbm, out_hbm, deg_hbm,
                    *scratch):
            _sc_body(True, x_hbm, src_hbm, dst_hbm, z2_hbm, z1_hbm,
                     out_hbm, deg_hbm, *scratch)
    else:
        @kern
        def sc_pass(x_hbm, src_hbm, dst_hbm, z2_hbm, out_hbm, *scratch):
            _sc_body(False, x_hbm, src_hbm, dst_hbm, z2_hbm, None,
                     out_hbm, None, *scratch)

    return sc_pass

BR = 1024
NB = NP // BR


def _dense_body(sa, sb, da, db, x, wl, wr, b, o, *, relu):
    deg = jnp.maximum(da[...] + db[...], 1.0)
    agg = (sa[...] + sb[...]) * (1.0 / deg)[:, None]
    y = jnp.dot(agg, wl[...], preferred_element_type=jnp.float32)
    y = y + jnp.dot(x[...], wr[...], preferred_element_type=jnp.float32)
    y = y + b[...]
    o[...] = jnp.maximum(y, 0.0) if relu else y


def _dense(summed, deg, xin, WlT, WrT, b, relu):
    return pl.pallas_call(
        functools.partial(_dense_body, relu=relu),
        grid=(NB,),
        in_specs=[
            pl.BlockSpec((BR, D), lambda i: (i, 0)),
            pl.BlockSpec((BR, D), lambda i: (i + NB, 0)),
            pl.BlockSpec((BR,), lambda i: (i,)),
            pl.BlockSpec((BR,), lambda i: (i + NB,)),
            pl.BlockSpec((BR, D), lambda i: (i, 0)),
            pl.BlockSpec((D, D), lambda i: (0, 0)),
            pl.BlockSpec((D, D), lambda i: (0, 0)),
            pl.BlockSpec((1, D), lambda i: (0, 0)),
        ],
        out_specs=pl.BlockSpec((BR, D), lambda i: (i, 0)),
        out_shape=jax.ShapeDtypeStruct((NP, D), jnp.float32),
    )(summed, summed, deg, deg, xin, WlT, WrT, b)


def kernel(x, edge_index, W1_l, b1, W1_r, W2_l, b2, W2_r):
    src = jnp.pad(edge_index[0].astype(jnp.int32).reshape(NW, EPW),
                  ((0, 0), (0, PADR * B - EPW))).reshape(NW, PADR, B)
    dst = jnp.pad(edge_index[1].astype(jnp.int32).reshape(NW, EPW),
                  ((0, 0), (0, PADR * B - EPW)),
                  constant_values=NP - 1).reshape(NW, PADR, B)
    z2 = jnp.zeros((RPT, D), jnp.float32)
    z1 = jnp.zeros((RPT,), jnp.float32)
    x_pad = jnp.pad(x, ((0, NP - N), (0, 0)))

    summed1, deg = _make_sc_pass(True)(x, src, dst, z2, z1)
    h = _dense(summed1, deg, x_pad, W1_l.T, W1_r.T, b1.reshape(1, D),
               relu=True)
    (summed2,) = _make_sc_pass(False)(h, src, dst, z2)
    out = _dense(summed2, deg, h, W2_l.T, W2_r.T, b2.reshape(1, D),
                 relu=False)
    return out[:N]


# speedup vs baseline: 2.0469x; 2.0469x over previous
import functools

import jax
import jax.numpy as jnp
from jax import lax
from jax.experimental import pallas as pl
from jax.experimental.pallas import tpu as pltpu
from jax.experimental.pallas import tpu_sc as plsc

N = 10000
E = 320000
D = 128
NP = 10240

NC = 2
NS = 16
NW = NC * NS
EPW = E // NW
B = 80
CH = EPW // B
HB = B // 2
RPT = NP // NS

def _sc_body(with_deg, x_hbm, src_hbm, dst_hbm, z2_hbm, z1_hbm,
             out_hbm, deg_hbm, src_v, dst_v, rows_v, ones_v,
             acc_s, deg_s, sem):
    c = lax.axis_index("c")
    s = lax.axis_index("s")
    wid = s * NC + c
    row0 = s * RPT

    pltpu.sync_copy(z2_hbm, acc_s.at[pl.ds(row0, RPT)])
    if with_deg:
        pltpu.sync_copy(z1_hbm, deg_s.at[pl.ds(row0, RPT)])
        for i in range(B // 16):
            ones_v[pl.ds(i * 16, 16)] = jnp.ones((16,), jnp.float32)

    pltpu.sync_copy(src_hbm.at[wid], src_v)
    pltpu.sync_copy(dst_hbm.at[wid], dst_v)
    plsc.subcore_barrier()

    def chunk(i, carry):
        ga = pltpu.async_copy(x_hbm.at[src_v.at[i, pl.ds(0, HB)]],
                              rows_v.at[pl.ds(0, HB)], sem)
        gb = pltpu.async_copy(x_hbm.at[src_v.at[i, pl.ds(HB, HB)]],
                              rows_v.at[pl.ds(HB, HB)], sem)
        ga.wait()
        gb.wait()
        sa = pltpu.async_copy(rows_v.at[pl.ds(0, HB)],
                              acc_s.at[dst_v.at[i, pl.ds(0, HB)]], sem,
                              add=True)
        sb = pltpu.async_copy(rows_v.at[pl.ds(HB, HB)],
                              acc_s.at[dst_v.at[i, pl.ds(HB, HB)]], sem,
                              add=True)
        if with_deg:
            pltpu.sync_copy(ones_v, deg_s.at[dst_v.at[i]], add=True)
        sa.wait()
        sb.wait()
        return carry

    lax.fori_loop(0, CH, chunk, 0)
    plsc.subcore_barrier()

    out0 = c * NP + row0
    pltpu.sync_copy(acc_s.at[pl.ds(row0, RPT)], out_hbm.at[pl.ds(out0, RPT)])
    if with_deg:
        pltpu.sync_copy(deg_s.at[pl.ds(row0, RPT)], deg_hbm.at[pl.ds(out0, RPT)])


@functools.lru_cache(maxsize=None)
def _make_sc_pass(with_deg):
    mesh = plsc.VectorSubcoreMesh(core_axis_name="c", subcore_axis_name="s")
    out_type = [jax.ShapeDtypeStruct((NC * NP, D), jnp.float32)]
    if with_deg:
        out_type.append(jax.ShapeDtypeStruct((NC * NP,), jnp.float32))
    kern = functools.partial(
        pl.kernel,
        mesh=mesh,
        out_type=out_type,
        scratch_types=[
            pltpu.VMEM((CH, B), jnp.int32),
            pltpu.VMEM((CH, B), jnp.int32),
            pltpu.VMEM((B, D), jnp.float32),
            pltpu.VMEM((B,), jnp.float32),
            pltpu.VMEM_SHARED((NP, D), jnp.float32),
            pltpu.VMEM_SHARED((NP,), jnp.float32),
            pltpu.SemaphoreType.DMA,
        ],
    )

    if with_deg:
        @kern
        def sc_pass(x_hbm, src_hbm, dst_hbm, z2_hbm, z1_hbm, out_hbm, deg_hbm,
                    *scratch):
            _sc_body(True, x_hbm, src_hbm, dst_hbm, z2_hbm, z1_hbm,
                     out_hbm, deg_hbm, *scratch)
    else:
        @kern
        def sc_pass(x_hbm, src_hbm, dst_hbm, z2_hbm, out_hbm, *scratch):
            _sc_body(False, x_hbm, src_hbm, dst_hbm, z2_hbm, None,
                     out_hbm, None, *scratch)

    return sc_pass

BR = 1024
NB = NP // BR


def _dense_body(sa, sb, da, db, x, wl, wr, b, o, *, relu):
    deg = jnp.maximum(da[...] + db[...], 1.0)
    agg = (sa[...] + sb[...]) * (1.0 / deg)[:, None]
    y = jnp.dot(agg, wl[...], preferred_element_type=jnp.float32)
    y = y + jnp.dot(x[...], wr[...], preferred_element_type=jnp.float32)
    y = y + b[...]
    o[...] = jnp.maximum(y, 0.0) if relu else y


def _dense(summed, deg, xin, WlT, WrT, b, relu):
    return pl.pallas_call(
        functools.partial(_dense_body, relu=relu),
        grid=(NB,),
        in_specs=[
            pl.BlockSpec((BR, D), lambda i: (i, 0)),
            pl.BlockSpec((BR, D), lambda i: (i + NB, 0)),
            pl.BlockSpec((BR,), lambda i: (i,)),
            pl.BlockSpec((BR,), lambda i: (i + NB,)),
            pl.BlockSpec((BR, D), lambda i: (i, 0)),
            pl.BlockSpec((D, D), lambda i: (0, 0)),
            pl.BlockSpec((D, D), lambda i: (0, 0)),
            pl.BlockSpec((1, D), lambda i: (0, 0)),
        ],
        out_specs=pl.BlockSpec((BR, D), lambda i: (i, 0)),
        out_shape=jax.ShapeDtypeStruct((NP, D), jnp.float32),
    )(summed, summed, deg, deg, xin, WlT, WrT, b)


def kernel(x, edge_index, W1_l, b1, W1_r, W2_l, b2, W2_r):
    src = edge_index[0].astype(jnp.int32).reshape(NW, CH, B)
    dst = edge_index[1].astype(jnp.int32).reshape(NW, CH, B)
    z2 = jnp.zeros((RPT, D), jnp.float32)
    z1 = jnp.zeros((RPT,), jnp.float32)
    x_pad = jnp.pad(x, ((0, NP - N), (0, 0)))

    summed1, deg = _make_sc_pass(True)(x, src, dst, z2, z1)
    h = _dense(summed1, deg, x_pad, W1_l.T, W1_r.T, b1.reshape(1, D),
               relu=True)
    (summed2,) = _make_sc_pass(False)(h, src, dst, z2)
    out = _dense(summed2, deg, h, W2_l.T, W2_r.T, b2.reshape(1, D),
                 relu=False)
    return out[:N]


# speedup vs baseline: 2.2185x; 1.0838x over previous
import functools

import jax
import jax.numpy as jnp
from jax import lax
from jax.experimental import pallas as pl
from jax.experimental.pallas import tpu as pltpu
from jax.experimental.pallas import tpu_sc as plsc

N = 10000
E = 320000
D = 128
NP = 10240

NC = 2
NS = 16
NW = NC * NS
EPW = E // NW
B = 80
CH = EPW // B
HB = B // 2
RPT = NP // NS

def _sc_body(with_deg, x_hbm, src_hbm, dst_hbm, z2_hbm, z1_hbm,
             out_hbm, deg_hbm, src_v, dst_v, rows_v, ones_v,
             acc_s, deg_s, semg0, semg1, sems0, sems1):
    c = lax.axis_index("c")
    s = lax.axis_index("s")
    wid = s * NC + c
    row0 = s * RPT

    pltpu.sync_copy(z2_hbm, acc_s.at[pl.ds(row0, RPT)])
    if with_deg:
        pltpu.sync_copy(z1_hbm, deg_s.at[pl.ds(row0, RPT)])
        for i in range(B // 16):
            ones_v[pl.ds(i * 16, 16)] = jnp.ones((16,), jnp.float32)

    pltpu.sync_copy(src_hbm.at[wid], src_v)
    pltpu.sync_copy(dst_hbm.at[wid], dst_v)
    plsc.subcore_barrier()

    def gth(i, h):
        return pltpu.async_copy(
            x_hbm.at[src_v.at[i, pl.ds(h * HB, HB)]],
            rows_v.at[pl.ds(h * HB, HB)], semg0 if h == 0 else semg1)

    def sct(i, h):
        return pltpu.async_copy(
            rows_v.at[pl.ds(h * HB, HB)],
            acc_s.at[dst_v.at[i, pl.ds(h * HB, HB)]],
            sems0 if h == 0 else sems1, add=True)

    def two_chunks(i0, carry):
        ga = gth(i0, 0)
        gb = gth(i0, 1)
        ga.wait()
        sa = sct(i0, 0)
        gb.wait()
        sb = sct(i0, 1)
        if with_deg:
            pltpu.sync_copy(ones_v, deg_s.at[dst_v.at[i0]], add=True)
        sa.wait()
        ga = gth(i0 + 1, 0)
        sb.wait()
        gb = gth(i0 + 1, 1)
        ga.wait()
        sa = sct(i0 + 1, 0)
        gb.wait()
        sb = sct(i0 + 1, 1)
        if with_deg:
            pltpu.sync_copy(ones_v, deg_s.at[dst_v.at[i0 + 1]], add=True)
        sa.wait()
        sb.wait()
        return carry

    lax.fori_loop(0, (CH - 1) // 2, lambda q, c: two_chunks(2 * q, c), 0)
    ga = gth(CH - 1, 0)
    gb = gth(CH - 1, 1)
    ga.wait()
    sa = sct(CH - 1, 0)
    gb.wait()
    sb = sct(CH - 1, 1)
    if with_deg:
        pltpu.sync_copy(ones_v, deg_s.at[dst_v.at[CH - 1]], add=True)
    sa.wait()
    sb.wait()
    plsc.subcore_barrier()

    out0 = c * NP + row0
    pltpu.sync_copy(acc_s.at[pl.ds(row0, RPT)], out_hbm.at[pl.ds(out0, RPT)])
    if with_deg:
        pltpu.sync_copy(deg_s.at[pl.ds(row0, RPT)], deg_hbm.at[pl.ds(out0, RPT)])


@functools.lru_cache(maxsize=None)
def _make_sc_pass(with_deg):
    mesh = plsc.VectorSubcoreMesh(core_axis_name="c", subcore_axis_name="s")
    out_type = [jax.ShapeDtypeStruct((NC * NP, D), jnp.float32)]
    if with_deg:
        out_type.append(jax.ShapeDtypeStruct((NC * NP,), jnp.float32))
    kern = functools.partial(
        pl.kernel,
        mesh=mesh,
        out_type=out_type,
        scratch_types=[
            pltpu.VMEM((CH, B), jnp.int32),
            pltpu.VMEM((CH, B), jnp.int32),
            pltpu.VMEM((B, D), jnp.float32),
            pltpu.VMEM((B,), jnp.float32),
            pltpu.VMEM_SHARED((NP, D), jnp.float32),
            pltpu.VMEM_SHARED((NP,), jnp.float32),
            pltpu.SemaphoreType.DMA,
            pltpu.SemaphoreType.DMA,
            pltpu.SemaphoreType.DMA,
            pltpu.SemaphoreType.DMA,
        ],
    )

    if with_deg:
        @kern
        def sc_pass(x_hbm, src_hbm, dst_hbm, z2_hbm, z1_hbm, out_hbm, deg_hbm,
                    *scratch):
            _sc_body(True, x_hbm, src_hbm, dst_hbm, z2_hbm, z1_hbm,
                     out_hbm, deg_hbm, *scratch)
    else:
        @kern
        def sc_pass(x_hbm, src_hbm, dst_hbm, z2_hbm, out_hbm, *scratch):
            _sc_body(False, x_hbm, src_hbm, dst_hbm, z2_hbm, None,
                     out_hbm, None, *scratch)

    return sc_pass

BR = 1024
NB = NP // BR


def _dense_body(sa, sb, da, db, x, wl, wr, b, o, *, relu):
    deg = jnp.maximum(da[...] + db[...], 1.0)
    agg = (sa[...] + sb[...]) * (1.0 / deg)[:, None]
    y = jnp.dot(agg, wl[...], preferred_element_type=jnp.float32)
    y = y + jnp.dot(x[...], wr[...], preferred_element_type=jnp.float32)
    y = y + b[...]
    o[...] = jnp.maximum(y, 0.0) if relu else y


def _dense(summed, deg, xin, WlT, WrT, b, relu):
    return pl.pallas_call(
        functools.partial(_dense_body, relu=relu),
        grid=(NB,),
        in_specs=[
            pl.BlockSpec((BR, D), lambda i: (i, 0)),
            pl.BlockSpec((BR, D), lambda i: (i + NB, 0)),
            pl.BlockSpec((BR,), lambda i: (i,)),
            pl.BlockSpec((BR,), lambda i: (i + NB,)),
            pl.BlockSpec((BR, D), lambda i: (i, 0)),
            pl.BlockSpec((D, D), lambda i: (0, 0)),
            pl.BlockSpec((D, D), lambda i: (0, 0)),
            pl.BlockSpec((1, D), lambda i: (0, 0)),
        ],
        out_specs=pl.BlockSpec((BR, D), lambda i: (i, 0)),
        out_shape=jax.ShapeDtypeStruct((NP, D), jnp.float32),
    )(summed, summed, deg, deg, xin, WlT, WrT, b)


def kernel(x, edge_index, W1_l, b1, W1_r, W2_l, b2, W2_r):
    src = edge_index[0].astype(jnp.int32).reshape(NW, CH, B)
    dst = edge_index[1].astype(jnp.int32).reshape(NW, CH, B)
    z2 = jnp.zeros((RPT, D), jnp.float32)
    z1 = jnp.zeros((RPT,), jnp.float32)
    x_pad = jnp.pad(x, ((0, NP - N), (0, 0)))

    summed1, deg = _make_sc_pass(True)(x, src, dst, z2, z1)
    h = _dense(summed1, deg, x_pad, W1_l.T, W1_r.T, b1.reshape(1, D),
               relu=True)
    (summed2,) = _make_sc_pass(False)(h, src, dst, z2)
    out = _dense(summed2, deg, h, W2_l.T, W2_r.T, b2.reshape(1, D),
                 relu=False)
    return out[:N]


# speedup vs baseline: 2.2542x; 1.0161x over previous
import functools

import jax
import jax.numpy as jnp
from jax import lax
from jax.experimental import pallas as pl
from jax.experimental.pallas import tpu as pltpu
from jax.experimental.pallas import tpu_sc as plsc

N = 10000
E = 320000
D = 128
NP = 10240

NC = 2
NS = 16
NW = NC * NS
EPW = E // NW
B = 80
CH = EPW // B
HB = B // 2
RPT = NP // NS

def _sc_body(with_deg, x_hbm, src_hbm, dst_hbm, z2_hbm, z1_hbm,
             out_hbm, deg_hbm, src_v, dst_v, rows_v, ones_v,
             acc_s, deg_s, semg0, semg1, sems0, sems1):
    c = lax.axis_index("c")
    s = lax.axis_index("s")
    wid = s * NC + c
    row0 = s * RPT

    z = pltpu.async_copy(z2_hbm, acc_s.at[pl.ds(row0, RPT)], semg0)
    if with_deg:
        zd = pltpu.async_copy(z1_hbm, deg_s.at[pl.ds(row0, RPT)], semg1)
        for i in range(B // 16):
            ones_v[pl.ds(i * 16, 16)] = jnp.ones((16,), jnp.float32)
    pltpu.sync_copy(src_hbm.at[wid], src_v)
    pltpu.sync_copy(dst_hbm.at[wid], dst_v)
    z.wait()
    if with_deg:
        zd.wait()
    plsc.subcore_barrier()

    def gth(i, h):
        return pltpu.async_copy(
            x_hbm.at[src_v.at[i, pl.ds(h * HB, HB)]],
            rows_v.at[pl.ds(h * HB, HB)], semg0 if h == 0 else semg1)

    def sct(i, h):
        return pltpu.async_copy(
            rows_v.at[pl.ds(h * HB, HB)],
            acc_s.at[dst_v.at[i, pl.ds(h * HB, HB)]],
            sems0 if h == 0 else sems1, add=True)

    def run_chunks(i0, n):
        ga = gth(i0, 0)
        gb = gth(i0, 1)
        sa = sb = None
        for k in range(n):
            i = i0 + k
            ga.wait()
            sa = sct(i, 0)
            gb.wait()
            sb = sct(i, 1)
            if with_deg:
                pltpu.sync_copy(ones_v, deg_s.at[dst_v.at[i]], add=True)
            if k + 1 < n:
                sa.wait()
                ga = gth(i + 1, 0)
                sb.wait()
                gb = gth(i + 1, 1)
        sa.wait()
        sb.wait()

    U = 5
    lax.fori_loop(0, CH // U, lambda q, c: (run_chunks(U * q, U), c)[1], 0)
    plsc.subcore_barrier()

    out0 = c * NP + row0
    pltpu.sync_copy(acc_s.at[pl.ds(row0, RPT)], out_hbm.at[pl.ds(out0, RPT)])
    if with_deg:
        pltpu.sync_copy(deg_s.at[pl.ds(row0, RPT)], deg_hbm.at[pl.ds(out0, RPT)])


@functools.lru_cache(maxsize=None)
def _make_sc_pass(with_deg):
    mesh = plsc.VectorSubcoreMesh(core_axis_name="c", subcore_axis_name="s")
    out_type = [jax.ShapeDtypeStruct((NC * NP, D), jnp.float32)]
    if with_deg:
        out_type.append(jax.ShapeDtypeStruct((NC * NP,), jnp.float32))
    kern = functools.partial(
        pl.kernel,
        mesh=mesh,
        out_type=out_type,
        scratch_types=[
            pltpu.VMEM((CH, B), jnp.int32),
            pltpu.VMEM((CH, B), jnp.int32),
            pltpu.VMEM((B, D), jnp.float32),
            pltpu.VMEM((B,), jnp.float32),
            pltpu.VMEM_SHARED((NP, D), jnp.float32),
            pltpu.VMEM_SHARED((NP,), jnp.float32),
            pltpu.SemaphoreType.DMA,
            pltpu.SemaphoreType.DMA,
            pltpu.SemaphoreType.DMA,
            pltpu.SemaphoreType.DMA,
        ],
    )

    if with_deg:
        @kern
        def sc_pass(x_hbm, src_hbm, dst_hbm, z2_hbm, z1_hbm, out_hbm, deg_hbm,
                    *scratch):
            _sc_body(True, x_hbm, src_hbm, dst_hbm, z2_hbm, z1_hbm,
                     out_hbm, deg_hbm, *scratch)
    else:
        @kern
        def sc_pass(x_hbm, src_hbm, dst_hbm, z2_hbm, out_hbm, *scratch):
            _sc_body(False, x_hbm, src_hbm, dst_hbm, z2_hbm, None,
                     out_hbm, None, *scratch)

    return sc_pass

BR = 1024
NB = NP // BR


def _dense_body(sa, sb, da, db, x, wl, wr, b, o, *, relu):
    deg = jnp.maximum(da[...] + db[...], 1.0)
    agg = (sa[...] + sb[...]) * (1.0 / deg)[:, None]
    y = jnp.dot(agg, wl[...], preferred_element_type=jnp.float32)
    y = y + jnp.dot(x[...], wr[...], preferred_element_type=jnp.float32)
    y = y + b[...]
    o[...] = jnp.maximum(y, 0.0) if relu else y


def _dense(summed, deg, xin, WlT, WrT, b, relu):
    return pl.pallas_call(
        functools.partial(_dense_body, relu=relu),
        grid=(NB,),
        in_specs=[
            pl.BlockSpec((BR, D), lambda i: (i, 0)),
            pl.BlockSpec((BR, D), lambda i: (i + NB, 0)),
            pl.BlockSpec((BR,), lambda i: (i,)),
            pl.BlockSpec((BR,), lambda i: (i + NB,)),
            pl.BlockSpec((BR, D), lambda i: (i, 0)),
            pl.BlockSpec((D, D), lambda i: (0, 0)),
            pl.BlockSpec((D, D), lambda i: (0, 0)),
            pl.BlockSpec((1, D), lambda i: (0, 0)),
        ],
        out_specs=pl.BlockSpec((BR, D), lambda i: (i, 0)),
        out_shape=jax.ShapeDtypeStruct((NP, D), jnp.float32),
    )(summed, summed, deg, deg, xin, WlT, WrT, b)


def kernel(x, edge_index, W1_l, b1, W1_r, W2_l, b2, W2_r):
    src = edge_index[0].astype(jnp.int32).reshape(NW, CH, B)
    dst = edge_index[1].astype(jnp.int32).reshape(NW, CH, B)
    z2 = jnp.zeros((RPT, D), jnp.float32)
    z1 = jnp.zeros((RPT,), jnp.float32)
    x_pad = jnp.pad(x, ((0, NP - N), (0, 0)))

    summed1, deg = _make_sc_pass(True)(x, src, dst, z2, z1)
    h = _dense(summed1, deg, x_pad, W1_l.T, W1_r.T, b1.reshape(1, D),
               relu=True)
    (summed2,) = _make_sc_pass(False)(h, src, dst, z2)
    out = _dense(summed2, deg, h, W2_l.T, W2_r.T, b2.reshape(1, D),
                 relu=False)
    return out[:N]


# speedup vs baseline: 2.2892x; 1.0156x over previous
import functools

import jax
import jax.numpy as jnp
from jax import lax
from jax.experimental import pallas as pl
from jax.experimental.pallas import tpu as pltpu
from jax.experimental.pallas import tpu_sc as plsc

N = 10000
E = 320000
D = 128
NP = 10240

NC = 2
NS = 16
NW = NC * NS
EPW = E // NW
B = 80
CH = EPW // B
HB = B // 2
RPT = NP // NS

def _sc_body(with_deg, x_hbm, src_hbm, dst_hbm, z2_hbm, z1_hbm,
             out_hbm, deg_hbm, src_v, dst_v, rows_v, ones_v,
             acc_s, deg_s, semg0, semg1, sems0, sems1, semd):
    c = lax.axis_index("c")
    s = lax.axis_index("s")
    wid = s * NC + c
    row0 = s * RPT

    z = pltpu.async_copy(z2_hbm, acc_s.at[pl.ds(row0, RPT)], semg0)
    if with_deg:
        zd = pltpu.async_copy(z1_hbm, deg_s.at[pl.ds(row0, RPT)], semg1)
        for i in range(B // 16):
            ones_v[pl.ds(i * 16, 16)] = jnp.ones((16,), jnp.float32)
    pltpu.sync_copy(src_hbm.at[wid], src_v)
    pltpu.sync_copy(dst_hbm.at[wid], dst_v)
    z.wait()
    if with_deg:
        zd.wait()
    plsc.subcore_barrier()

    def gth(i, h):
        return pltpu.async_copy(
            x_hbm.at[src_v.at[i, pl.ds(h * HB, HB)]],
            rows_v.at[pl.ds(h * HB, HB)], semg0 if h == 0 else semg1)

    def sct(i, h):
        return pltpu.async_copy(
            rows_v.at[pl.ds(h * HB, HB)],
            acc_s.at[dst_v.at[i, pl.ds(h * HB, HB)]],
            sems0 if h == 0 else sems1, add=True)

    def run_chunks(i0, n):
        ga = gth(i0, 0)
        gb = gth(i0, 1)
        sa = sb = None
        sd = None
        for k in range(n):
            i = i0 + k
            ga.wait()
            sa = sct(i, 0)
            gb.wait()
            sb = sct(i, 1)
            if with_deg:
                if sd is not None:
                    sd.wait()
                sd = pltpu.async_copy(ones_v, deg_s.at[dst_v.at[i]],
                                      semd, add=True)
            if k + 1 < n:
                sa.wait()
                ga = gth(i + 1, 0)
                sb.wait()
                gb = gth(i + 1, 1)
        sa.wait()
        sb.wait()
        if with_deg:
            sd.wait()

    U = 5
    lax.fori_loop(0, CH // U, lambda q, c: (run_chunks(U * q, U), c)[1], 0)
    plsc.subcore_barrier()

    out0 = c * NP + row0
    pltpu.sync_copy(acc_s.at[pl.ds(row0, RPT)], out_hbm.at[pl.ds(out0, RPT)])
    if with_deg:
        pltpu.sync_copy(deg_s.at[pl.ds(row0, RPT)], deg_hbm.at[pl.ds(out0, RPT)])


@functools.lru_cache(maxsize=None)
def _make_sc_pass(with_deg):
    mesh = plsc.VectorSubcoreMesh(core_axis_name="c", subcore_axis_name="s")
    out_type = [jax.ShapeDtypeStruct((NC * NP, D), jnp.float32)]
    if with_deg:
        out_type.append(jax.ShapeDtypeStruct((NC * NP,), jnp.float32))
    kern = functools.partial(
        pl.kernel,
        mesh=mesh,
        out_type=out_type,
        scratch_types=[
            pltpu.VMEM((CH, B), jnp.int32),
            pltpu.VMEM((CH, B), jnp.int32),
            pltpu.VMEM((B, D), jnp.float32),
            pltpu.VMEM((B,), jnp.float32),
            pltpu.VMEM_SHARED((NP, D), jnp.float32),
            pltpu.VMEM_SHARED((NP,), jnp.float32),
            pltpu.SemaphoreType.DMA,
            pltpu.SemaphoreType.DMA,
            pltpu.SemaphoreType.DMA,
            pltpu.SemaphoreType.DMA,
            pltpu.SemaphoreType.DMA,
        ],
    )

    if with_deg:
        @kern
        def sc_pass(x_hbm, src_hbm, dst_hbm, z2_hbm, z1_hbm, out_hbm, deg_hbm,
                    *scratch):
            _sc_body(True, x_hbm, src_hbm, dst_hbm, z2_hbm, z1_hbm,
                     out_hbm, deg_hbm, *scratch)
    else:
        @kern
        def sc_pass(x_hbm, src_hbm, dst_hbm, z2_hbm, out_hbm, *scratch):
            _sc_body(False, x_hbm, src_hbm, dst_hbm, z2_hbm, None,
                     out_hbm, None, *scratch)

    return sc_pass

BR = 1024
NB = NP // BR


def _dense_body(sa, sb, da, db, x, wl, wr, b, o, *, relu):
    deg = jnp.maximum(da[...] + db[...], 1.0)
    agg = (sa[...] + sb[...]) * (1.0 / deg)[:, None]
    y = jnp.dot(agg, wl[...], preferred_element_type=jnp.float32)
    y = y + jnp.dot(x[...], wr[...], preferred_element_type=jnp.float32)
    y = y + b[...]
    o[...] = jnp.maximum(y, 0.0) if relu else y


def _dense(summed, deg, xin, WlT, WrT, b, relu):
    return pl.pallas_call(
        functools.partial(_dense_body, relu=relu),
        grid=(NB,),
        in_specs=[
            pl.BlockSpec((BR, D), lambda i: (i, 0)),
            pl.BlockSpec((BR, D), lambda i: (i + NB, 0)),
            pl.BlockSpec((BR,), lambda i: (i,)),
            pl.BlockSpec((BR,), lambda i: (i + NB,)),
            pl.BlockSpec((BR, D), lambda i: (i, 0)),
            pl.BlockSpec((D, D), lambda i: (0, 0)),
            pl.BlockSpec((D, D), lambda i: (0, 0)),
            pl.BlockSpec((1, D), lambda i: (0, 0)),
        ],
        out_specs=pl.BlockSpec((BR, D), lambda i: (i, 0)),
        out_shape=jax.ShapeDtypeStruct((NP, D), jnp.float32),
    )(summed, summed, deg, deg, xin, WlT, WrT, b)


def kernel(x, edge_index, W1_l, b1, W1_r, W2_l, b2, W2_r):
    src = edge_index[0].astype(jnp.int32).reshape(NW, CH, B)
    dst = edge_index[1].astype(jnp.int32).reshape(NW, CH, B)
    z2 = jnp.zeros((RPT, D), jnp.float32)
    z1 = jnp.zeros((RPT,), jnp.float32)

    summed1, deg = _make_sc_pass(True)(x, src, dst, z2, z1)
    h = _dense(summed1, deg, x, W1_l.T, W1_r.T, b1.reshape(1, D),
               relu=True)
    (summed2,) = _make_sc_pass(False)(h, src, dst, z2)
    out = _dense(summed2, deg, h, W2_l.T, W2_r.T, b2.reshape(1, D),
                 relu=False)
    return out[:N]
